# baseline (speedup 1.0000x reference)
import functools

import jax
import jax.numpy as jnp
from jax import lax
from jax.experimental import pallas as pl
from jax.experimental.pallas import tpu as pltpu
from jax.experimental.pallas import tpu_sc as plsc

N = 10000
E = 320000
G = 16
IN_DIM = 128
HID = 64
EMB = 32

NC, NS = 2, 16
NW = NC * NS
CH = 128
E_PAD = 327680
N_PAD = 10112
RPT = N_PAD // NS
CW = 8
NBC = 64


def _splat(v):
    return jnp.full((16,), v, jnp.int32)


def _make_prop(D):
    NG = D // CW
    NREP = NW // NG
    NCH = E_PAD // (NREP * CH)
    NBLK = NCH // NBC
    NVEC = NBC * CH // 16
    mesh = plsc.VectorSubcoreMesh(core_axis_name="c", subcore_axis_name="s")

    @functools.partial(
        pl.kernel,
        out_type=jax.ShapeDtypeStruct((NW, CW, N_PAD), jnp.float32),
        mesh=mesh,
        compiler_params=pltpu.CompilerParams(
            use_tc_tiling_on_sc=False, needs_layout_passes=False),
        scratch_types=[
            pltpu.VMEM((CW, N_PAD), jnp.float32),
            pltpu.VMEM((NBC, CH), jnp.int32),
            pltpu.VMEM((NBC, CH), jnp.int32),
            pltpu.VMEM((CH, CW), jnp.float32),
            pltpu.VMEM((CH, CW), jnp.float32),
            pltpu.SemaphoreType.DMA,
            pltpu.SemaphoreType.DMA,
        ],
    )
    def prop(g_hbm, src_hbm, dst_hbm, zeros_hbm, out_hbm, acc, sblk, dblk,
             v0, v1, sem0, sem1):
        c = lax.axis_index("c")
        s = lax.axis_index("s")
        wid = c * NS + s
        r = wid // NG
        k = wid % NG
        pltpu.sync_copy(zeros_hbm, acc)

        colv = lax.iota(jnp.int32, 16) & 7
        m8 = lax.iota(jnp.int32, 16) < 8
        kv = jnp.full((16,), k, jnp.int32)

        def compute_chunk(ci, vbuf):
            ci_s = _splat(ci)
            for e in range(CH):
                rowv = plsc.load_gather(dblk, [ci_s, _splat(e)])
                valv = plsc.load_gather(vbuf, [_splat(e), colv])
                plsc.addupdate_scatter(acc, [colv, rowv], valv, mask=m8)

        def blk_body(bi, carry):
            cb = bi * NBC
            pltpu.sync_copy(src_hbm.at[r].at[pl.ds(cb, NBC)], sblk)
            pltpu.sync_copy(dst_hbm.at[r].at[pl.ds(cb, NBC)], dblk)

            def tx_body(t, carry3):
                q = t * 16
                ci = q // CH
                off = q % CH
                v = sblk[ci, pl.ds(off, 16)]
                sblk[ci, pl.ds(off, 16)] = v * NG + kv
                return carry3

            lax.fori_loop(0, NVEC, tx_body, 0)
            pltpu.async_copy(g_hbm.at[sblk.at[0]], v0, sem0)

            def cj_body(cj, carry2):
                c0 = cj * 2
                c1 = c0 + 1
                pltpu.async_copy(g_hbm.at[sblk.at[c1]], v1, sem1)
                pltpu.make_async_copy(g_hbm.at[sblk.at[c0]], v0, sem0).wait()
                compute_chunk(c0, v0)

                @pl.when(cj < NBC // 2 - 1)
                def _():
                    pltpu.async_copy(g_hbm.at[sblk.at[c0 + 2]], v0, sem0)

                pltpu.make_async_copy(g_hbm.at[sblk.at[c1]], v1, sem1).wait()
                compute_chunk(c1, v1)
                return carry2

            lax.fori_loop(0, NBC // 2, cj_body, 0)
            return carry

        lax.fori_loop(0, NBLK, blk_body, 0)
        pltpu.sync_copy(acc, out_hbm.at[wid])

    return prop


_prop64 = _make_prop(HID)
_prop32 = _make_prop(EMB)

DEGW = 16
NCHD = E_PAD // (NW * CH)


def _make_deg():
    mesh = plsc.VectorSubcoreMesh(core_axis_name="c", subcore_axis_name="s")

    @functools.partial(
        pl.kernel,
        out_type=jax.ShapeDtypeStruct((NC, N_PAD, DEGW), jnp.float32),
        mesh=mesh,
        compiler_params=pltpu.CompilerParams(use_tc_tiling_on_sc=False),
        scratch_types=[
            pltpu.VMEM_SHARED((N_PAD, DEGW), jnp.float32),
            pltpu.VMEM((NCHD, CH), jnp.int32),
            pltpu.VMEM((CH, DEGW), jnp.float32),
            pltpu.SemaphoreType.DMA,
        ],
    )
    def deg(ones_hbm, dst_hbm, out_hbm, acc, idst, rows, sem):
        c = lax.axis_index("c")
        s = lax.axis_index("s")
        wid = c * NS + s
        base = s * RPT
        pltpu.sync_copy(dst_hbm.at[wid], idst)
        pltpu.sync_copy(ones_hbm.at[pl.ds(0, CH)], rows)
        pltpu.sync_copy(ones_hbm.at[pl.ds(base, RPT)],
                        acc.at[pl.ds(base, RPT)])
        plsc.subcore_barrier()

        def body(jo, carry):
            for u in range(8):
                pltpu.async_copy(rows, acc.at[idst.at[jo * 8 + u]], sem,
                                 add=True)
            for u in range(8):
                pltpu.make_async_copy(rows, acc.at[idst.at[jo * 8 + u]],
                                      sem).wait()
            return carry

        lax.fori_loop(0, NCHD // 8, body, 0)
        plsc.subcore_barrier()
        pltpu.sync_copy(acc.at[pl.ds(base, RPT)],
                        out_hbm.at[c].at[pl.ds(base, RPT)])

    return deg


_deg = _make_deg()


def _combine_t(parts, gT, D):
    NG = D // CW
    NREP = NW // NG
    cols = []
    for k in range(NG):
        p = parts[k]
        for rr in range(1, NREP):
            p = p + parts[rr * NG + k]
        cols.append(p)
    return jnp.concatenate(cols, axis=0) + gT


def _tc_first(x_ref, w_ref, d0_ref, d1_ref, g_ref, gt_ref, dinv_ref):
    deg = d0_ref[...] + d1_ref[...] - 1.0
    dinvT = lax.rsqrt(deg)
    dinv_ref[...] = dinvT
    gT = dinvT * lax.dot_general(
        w_ref[...], x_ref[...], (((0,), (1,)), ((), ())),
        preferred_element_type=jnp.float32)
    gt_ref[...] = gT
    g_ref[...] = gT.T


def _tc_mid(p_ref, gt_ref, dinv_ref, b_ref, w_ref, gn_ref, gnt_ref,
            *, d_in, d_out):
    parts = [p_ref[i] for i in range(NW)]
    pT = _combine_t(parts, gt_ref[...], d_in)
    dinvT = dinv_ref[...]
    hT = jnp.maximum(dinvT * pT + b_ref[...], 0.0)
    gT = dinvT * lax.dot_general(
        w_ref[...], hT, (((0,), (0,)), ((), ())),
        preferred_element_type=jnp.float32)
    gnt_ref[...] = gT
    gn_ref[...] = gT.T


def _tc_final(p_ref, gt_ref, dinv_ref, b_ref, batch_ref, out_ref):
    parts = [p_ref[i] for i in range(NW)]
    pT = _combine_t(parts, gt_ref[...], EMB)
    hT = dinv_ref[...] * pT + b_ref[...]
    gid = lax.broadcasted_iota(jnp.int32, (N_PAD, G), 1)
    mT = (batch_ref[...] == gid).astype(jnp.float32)
    sums = jnp.dot(hT, mT, preferred_element_type=jnp.float32)
    cnt = jnp.sum(mT, axis=0, keepdims=True)
    out_ref[...] = (sums / jnp.maximum(cnt, 1.0)).T


def kernel(x, edge_index, batch, W1, b1, W2, b2, W3, b3):
    f32 = jnp.float32
    src = edge_index[0].astype(jnp.int32)
    dst = edge_index[1].astype(jnp.int32)
    src_p = jnp.concatenate([src, jnp.zeros((E_PAD - E,), jnp.int32)])
    dst_p = jnp.concatenate([dst, jnp.full((E_PAD - E,), N, jnp.int32)])
    dst_deg = dst_p.reshape(NW, NCHD, CH)
    src_r4 = src_p.reshape(4, E_PAD // (4 * CH), CH)
    dst_r4 = dst_p.reshape(4, E_PAD // (4 * CH), CH)
    src_r8 = src_p.reshape(8, E_PAD // (8 * CH), CH)
    dst_r8 = dst_p.reshape(8, E_PAD // (8 * CH), CH)

    x_p = jnp.zeros((N_PAD, IN_DIM), f32).at[:N].set(x)
    batch_c = jnp.full((N_PAD, 1), G, jnp.int32).at[:N, 0].set(
        batch.astype(jnp.int32))
    ones = jnp.ones((N_PAD, DEGW), f32)
    zT = jnp.zeros((CW, N_PAD), f32)

    dparts = _deg(ones, dst_deg)
    d0 = dparts[0, :, 0].reshape(1, N_PAD)
    d1 = dparts[1, :, 0].reshape(1, N_PAD)

    g1, g1T, dinvT = pl.pallas_call(
        _tc_first,
        out_shape=[jax.ShapeDtypeStruct((N_PAD, HID), f32),
                   jax.ShapeDtypeStruct((HID, N_PAD), f32),
                   jax.ShapeDtypeStruct((1, N_PAD), f32)],
    )(x_p, W1, d0, d1)

    p1 = _prop64(g1.reshape(N_PAD * (HID // CW), CW), src_r4, dst_r4, zT)
    g2, g2T = pl.pallas_call(
        functools.partial(_tc_mid, d_in=HID, d_out=HID),
        out_shape=[jax.ShapeDtypeStruct((N_PAD, HID), f32),
                   jax.ShapeDtypeStruct((HID, N_PAD), f32)],
    )(p1, g1T, dinvT, b1.reshape(HID, 1), W2)

    p2 = _prop64(g2.reshape(N_PAD * (HID // CW), CW), src_r4, dst_r4, zT)
    g3, g3T = pl.pallas_call(
        functools.partial(_tc_mid, d_in=HID, d_out=EMB),
        out_shape=[jax.ShapeDtypeStruct((N_PAD, EMB), f32),
                   jax.ShapeDtypeStruct((EMB, N_PAD), f32)],
    )(p2, g2T, dinvT, b2.reshape(HID, 1), W3)

    p3 = _prop32(g3.reshape(N_PAD * (EMB // CW), CW), src_r8, dst_r8, zT)
    out = pl.pallas_call(
        _tc_final,
        out_shape=jax.ShapeDtypeStruct((G, EMB), f32),
    )(p3, g3T, dinvT, b3.reshape(EMB, 1), batch_c)
    return out


# speedup vs baseline: 1.6445x; 1.6445x over previous
import functools

import jax
import jax.numpy as jnp
from jax import lax
from jax.experimental import pallas as pl
from jax.experimental.pallas import tpu as pltpu
from jax.experimental.pallas import tpu_sc as plsc

N = 10000
E = 320000
G = 16
IN_DIM = 128
HID = 64
EMB = 32

NC, NS = 2, 16
NW = NC * NS
CH = 128
E_PAD = 327680
N_PAD = 10112
RPT = N_PAD // NS
CW = 8
NBC = 64
N_STR = N_PAD + 1


def _splat(v):
    return jnp.full((16,), v, jnp.int32)


def _make_prop(D):
    NG = D // CW
    NREP = NW // NG
    NCH = E_PAD // (NREP * CH)
    NBLK = NCH // NBC
    NVEC = NBC * CH // 16
    mesh = plsc.VectorSubcoreMesh(core_axis_name="c", subcore_axis_name="s")

    @functools.partial(
        pl.kernel,
        out_type=jax.ShapeDtypeStruct((NW, CW, N_STR), jnp.float32),
        mesh=mesh,
        compiler_params=pltpu.CompilerParams(
            use_tc_tiling_on_sc=False, needs_layout_passes=False),
        scratch_types=[
            pltpu.VMEM((CW, N_STR), jnp.float32),
            pltpu.VMEM((NBC, CH), jnp.int32),
            pltpu.VMEM((NBC, CH), jnp.int32),
            pltpu.VMEM((CH, CW), jnp.float32),
            pltpu.VMEM((CH, CW), jnp.float32),
            pltpu.SemaphoreType.DMA,
            pltpu.SemaphoreType.DMA,
        ],
    )
    def prop(g_hbm, src_hbm, dst_hbm, zeros_hbm, out_hbm, acc, sblk, dblk,
             v0, v1, sem0, sem1):
        c = lax.axis_index("c")
        s = lax.axis_index("s")
        wid = c * NS + s
        r = wid // NG
        k = wid % NG
        pltpu.sync_copy(zeros_hbm, acc)

        colv = lax.iota(jnp.int32, 16) & 7
        half = (lax.iota(jnp.int32, 16) >= 8).astype(jnp.int32)
        kv = jnp.full((16,), k, jnp.int32)
        pats = [_splat(2 * p) + half for p in range(8)]

        def compute_chunk(ci, vbuf):
            for gv in range(CH // 16):
                dstv = dblk[ci, pl.ds(gv * 16, 16)]
                gv16 = _splat(gv * 16)
                for p in range(8):
                    rowv = dstv.at[pats[p]].get(mode="promise_in_bounds")
                    vrow = pats[p] + gv16
                    valv = plsc.load_gather(vbuf, [vrow, colv])
                    plsc.addupdate_scatter(acc, [colv, rowv], valv)

        def blk_body(bi, carry):
            cb = bi * NBC
            pltpu.sync_copy(src_hbm.at[r].at[pl.ds(cb, NBC)], sblk)
            pltpu.sync_copy(dst_hbm.at[r].at[pl.ds(cb, NBC)], dblk)

            def tx_body(t, carry3):
                q = t * 16
                ci = q // CH
                off = q % CH
                v = sblk[ci, pl.ds(off, 16)]
                sblk[ci, pl.ds(off, 16)] = v * NG + kv
                return carry3

            lax.fori_loop(0, NVEC, tx_body, 0)
            pltpu.async_copy(g_hbm.at[sblk.at[0]], v0, sem0)

            def cj_body(cj, carry2):
                c0 = cj * 2
                c1 = c0 + 1
                pltpu.async_copy(g_hbm.at[sblk.at[c1]], v1, sem1)
                pltpu.make_async_copy(g_hbm.at[sblk.at[c0]], v0, sem0).wait()
                compute_chunk(c0, v0)

                @pl.when(cj < NBC // 2 - 1)
                def _():
                    pltpu.async_copy(g_hbm.at[sblk.at[c0 + 2]], v0, sem0)

                pltpu.make_async_copy(g_hbm.at[sblk.at[c1]], v1, sem1).wait()
                compute_chunk(c1, v1)
                return carry2

            lax.fori_loop(0, NBC // 2, cj_body, 0)
            return carry

        lax.fori_loop(0, NBLK, blk_body, 0)
        pltpu.sync_copy(acc, out_hbm.at[wid])

    return prop


_prop64 = _make_prop(HID)
_prop32 = _make_prop(EMB)

NCHD = E_PAD // (NW * CH)


def _make_deg():
    mesh = plsc.VectorSubcoreMesh(core_axis_name="c", subcore_axis_name="s")

    @functools.partial(
        pl.kernel,
        out_type=jax.ShapeDtypeStruct((NW, N_PAD), jnp.float32),
        mesh=mesh,
        compiler_params=pltpu.CompilerParams(
            use_tc_tiling_on_sc=False, needs_layout_passes=False),
        scratch_types=[
            pltpu.VMEM((N_PAD,), jnp.float32),
            pltpu.VMEM((NCHD, CH), jnp.int32),
        ],
    )
    def deg(dst_hbm, zeros_hbm, out_hbm, acc, idst):
        c = lax.axis_index("c")
        s = lax.axis_index("s")
        wid = c * NS + s
        pltpu.sync_copy(dst_hbm.at[wid], idst)
        pltpu.sync_copy(zeros_hbm, acc)
        onev = jnp.full((16,), 1.0, jnp.float32)

        def body(ci, carry):
            for u in range(CH // 16):
                dstv = idst[ci, pl.ds(u * 16, 16)]
                plsc.addupdate_scatter(acc, [dstv], onev)
            return carry

        lax.fori_loop(0, NCHD, body, 0)
        pltpu.sync_copy(acc, out_hbm.at[wid])

    return deg


_deg = _make_deg()


def _combine_t(parts, gT, D):
    NG = D // CW
    NREP = NW // NG
    cols = []
    for k in range(NG):
        p = parts[k]
        for rr in range(1, NREP):
            p = p + parts[rr * NG + k]
        cols.append(p[:, :N_PAD])
    return jnp.concatenate(cols, axis=0) + gT


def _tc_first(x_ref, w_ref, d_ref, g_ref, gt_ref, dinv_ref):
    deg = jnp.sum(d_ref[...], axis=0, keepdims=True) + 1.0
    dinvT = lax.rsqrt(deg)
    dinv_ref[...] = dinvT
    gT = dinvT * lax.dot_general(
        w_ref[...], x_ref[...], (((0,), (1,)), ((), ())),
        preferred_element_type=jnp.float32)
    gt_ref[...] = gT
    g_ref[...] = gT.T


def _tc_mid(p_ref, gt_ref, dinv_ref, b_ref, w_ref, gn_ref, gnt_ref,
            *, d_in, d_out):
    parts = [p_ref[i] for i in range(NW)]
    pT = _combine_t(parts, gt_ref[...], d_in)
    dinvT = dinv_ref[...]
    hT = jnp.maximum(dinvT * pT + b_ref[...], 0.0)
    gT = dinvT * lax.dot_general(
        w_ref[...], hT, (((0,), (0,)), ((), ())),
        preferred_element_type=jnp.float32)
    gnt_ref[...] = gT
    gn_ref[...] = gT.T


def _tc_final(p_ref, gt_ref, dinv_ref, b_ref, batch_ref, out_ref):
    parts = [p_ref[i] for i in range(NW)]
    pT = _combine_t(parts, gt_ref[...], EMB)
    hT = dinv_ref[...] * pT + b_ref[...]
    gid = lax.broadcasted_iota(jnp.int32, (N_PAD, G), 1)
    mT = (batch_ref[...] == gid).astype(jnp.float32)
    sums = jnp.dot(hT, mT, preferred_element_type=jnp.float32)
    cnt = jnp.sum(mT, axis=0, keepdims=True)
    out_ref[...] = (sums / jnp.maximum(cnt, 1.0)).T


def kernel(x, edge_index, batch, W1, b1, W2, b2, W3, b3):
    f32 = jnp.float32
    src = edge_index[0].astype(jnp.int32)
    dst = edge_index[1].astype(jnp.int32)
    src_p = jnp.concatenate([src, jnp.zeros((E_PAD - E,), jnp.int32)])
    dst_p = jnp.concatenate([dst, jnp.full((E_PAD - E,), N, jnp.int32)])
    dst_deg = dst_p.reshape(NW, NCHD, CH)
    src_r4 = src_p.reshape(4, E_PAD // (4 * CH), CH)
    dst_r4 = dst_p.reshape(4, E_PAD // (4 * CH), CH)
    src_r8 = src_p.reshape(8, E_PAD // (8 * CH), CH)
    dst_r8 = dst_p.reshape(8, E_PAD // (8 * CH), CH)

    x_p = jnp.zeros((N_PAD, IN_DIM), f32).at[:N].set(x)
    batch_c = jnp.full((N_PAD, 1), G, jnp.int32).at[:N, 0].set(
        batch.astype(jnp.int32))
    zT = jnp.zeros((CW, N_STR), f32)
    z_n = jnp.zeros((N_PAD,), f32)

    dparts = _deg(dst_deg, z_n)

    g1, g1T, dinvT = pl.pallas_call(
        _tc_first,
        out_shape=[jax.ShapeDtypeStruct((N_PAD, HID), f32),
                   jax.ShapeDtypeStruct((HID, N_PAD), f32),
                   jax.ShapeDtypeStruct((1, N_PAD), f32)],
    )(x_p, W1, dparts)

    p1 = _prop64(g1.reshape(N_PAD * (HID // CW), CW), src_r4, dst_r4, zT)
    g2, g2T = pl.pallas_call(
        functools.partial(_tc_mid, d_in=HID, d_out=HID),
        out_shape=[jax.ShapeDtypeStruct((N_PAD, HID), f32),
                   jax.ShapeDtypeStruct((HID, N_PAD), f32)],
    )(p1, g1T, dinvT, b1.reshape(HID, 1), W2)

    p2 = _prop64(g2.reshape(N_PAD * (HID // CW), CW), src_r4, dst_r4, zT)
    g3, g3T = pl.pallas_call(
        functools.partial(_tc_mid, d_in=HID, d_out=EMB),
        out_shape=[jax.ShapeDtypeStruct((N_PAD, EMB), f32),
                   jax.ShapeDtypeStruct((EMB, N_PAD), f32)],
    )(p2, g2T, dinvT, b2.reshape(HID, 1), W3)

    p3 = _prop32(g3.reshape(N_PAD * (EMB // CW), CW), src_r8, dst_r8, zT)
    out = pl.pallas_call(
        _tc_final,
        out_shape=jax.ShapeDtypeStruct((G, EMB), f32),
    )(p3, g3T, dinvT, b3.reshape(EMB, 1), batch_c)
    return out


# speedup vs baseline: 3.9874x; 2.4247x over previous
import functools

import jax
import jax.numpy as jnp
from jax import lax
from jax.experimental import pallas as pl
from jax.experimental.pallas import tpu as pltpu
from jax.experimental.pallas import tpu_sc as plsc

N = 10000
E = 320000
G = 16
IN_DIM = 128
HID = 64
EMB = 32

NC, NS = 2, 16
NW = NC * NS
CH = 128
NCHUNK = 80
EPT_PAD = NCHUNK * CH
E_PAD = EPT_PAD * NW
N_PAD = 10112
RPT = N_PAD // NS


def _make_prop(D):
    mesh = plsc.VectorSubcoreMesh(core_axis_name="c", subcore_axis_name="s")

    @functools.partial(
        pl.kernel,
        out_type=jax.ShapeDtypeStruct((NC, N_PAD, D), jnp.float32),
        mesh=mesh,
        compiler_params=pltpu.CompilerParams(use_tc_tiling_on_sc=False),
        scratch_types=[
            pltpu.VMEM_SHARED((N_PAD, D), jnp.float32),
            pltpu.VMEM((NCHUNK, CH), jnp.int32),
            pltpu.VMEM((NCHUNK, CH), jnp.int32),
            pltpu.VMEM((CH, D), jnp.float32),
            pltpu.VMEM((CH, D), jnp.float32),
            pltpu.SemaphoreType.DMA,
            pltpu.SemaphoreType.DMA,
            pltpu.SemaphoreType.DMA,
            pltpu.SemaphoreType.DMA,
        ],
    )
    def prop(g_hbm, src_hbm, dst_hbm, out_hbm, acc, isrc, idst,
             rows0, rows1, gs0, gs1, ss0, ss1):
        c = lax.axis_index("c")
        s = lax.axis_index("s")
        wid = c * NS + s
        base = s * RPT
        pltpu.sync_copy(src_hbm.at[wid], isrc)
        pltpu.sync_copy(dst_hbm.at[wid], idst)
        pltpu.sync_copy(g_hbm.at[pl.ds(base, RPT)], acc.at[pl.ds(base, RPT)])
        plsc.subcore_barrier()

        pltpu.async_copy(g_hbm.at[isrc.at[0]], rows0, gs0)
        pltpu.async_copy(g_hbm.at[isrc.at[1]], rows1, gs1)

        def body(jj, carry):
            j0 = jj * 2
            j1 = j0 + 1
            pltpu.make_async_copy(g_hbm.at[isrc.at[j0]], rows0, gs0).wait()
            pltpu.async_copy(rows0, acc.at[idst.at[j0]], ss0, add=True)
            pltpu.make_async_copy(g_hbm.at[isrc.at[j1]], rows1, gs1).wait()
            pltpu.async_copy(rows1, acc.at[idst.at[j1]], ss1, add=True)

            @pl.when(jj < NCHUNK // 2 - 1)
            def _():
                pltpu.make_async_copy(
                    rows0, acc.at[idst.at[j0]], ss0).wait()
                pltpu.async_copy(g_hbm.at[isrc.at[j0 + 2]], rows0, gs0)
                pltpu.make_async_copy(
                    rows1, acc.at[idst.at[j1]], ss1).wait()
                pltpu.async_copy(g_hbm.at[isrc.at[j1 + 2]], rows1, gs1)

            @pl.when(jj == NCHUNK // 2 - 1)
            def _():
                pltpu.make_async_copy(
                    rows0, acc.at[idst.at[j0]], ss0).wait()
                pltpu.make_async_copy(
                    rows1, acc.at[idst.at[j1]], ss1).wait()
            return carry

        lax.fori_loop(0, NCHUNK // 2, body, 0)
        plsc.subcore_barrier()
        pltpu.sync_copy(acc.at[pl.ds(base, RPT)],
                        out_hbm.at[c].at[pl.ds(base, RPT)])

    return prop


_prop64 = _make_prop(HID)
_prop32 = _make_prop(EMB)

def _make_deg():
    mesh = plsc.VectorSubcoreMesh(core_axis_name="c", subcore_axis_name="s")

    @functools.partial(
        pl.kernel,
        out_type=jax.ShapeDtypeStruct((NW, N_PAD), jnp.float32),
        mesh=mesh,
        compiler_params=pltpu.CompilerParams(
            use_tc_tiling_on_sc=False, needs_layout_passes=False),
        scratch_types=[
            pltpu.VMEM((N_PAD,), jnp.float32),
            pltpu.VMEM((NCHUNK, CH), jnp.int32),
        ],
    )
    def deg(dst_hbm, zeros_hbm, out_hbm, acc, idst):
        c = lax.axis_index("c")
        s = lax.axis_index("s")
        wid = c * NS + s
        pltpu.sync_copy(dst_hbm.at[wid], idst)
        pltpu.sync_copy(zeros_hbm, acc)
        onev = jnp.full((16,), 1.0, jnp.float32)

        def body(ci, carry):
            for u in range(CH // 16):
                dstv = idst[ci, pl.ds(u * 16, 16)]
                plsc.addupdate_scatter(acc, [dstv], onev)
            return carry

        lax.fori_loop(0, NCHUNK, body, 0)
        pltpu.sync_copy(acc, out_hbm.at[wid])

    return deg


_deg = _make_deg()


def _tc_first(x_ref, w_ref, dt_ref, g_ref, dinv_ref):
    deg = jnp.sum(dt_ref[...], axis=1, keepdims=True) + 1.0
    dinv = lax.rsqrt(deg)
    dinv_ref[...] = dinv
    g_ref[...] = dinv * jnp.dot(x_ref[...], w_ref[...],
                                preferred_element_type=jnp.float32)


def _tc_mid(a0_ref, a1_ref, g_ref, dinv_ref, b_ref, w_ref, gn_ref):
    p = a0_ref[...] + a1_ref[...] - g_ref[...]
    h = jnp.maximum(dinv_ref[...] * p + b_ref[...], 0.0)
    gn_ref[...] = dinv_ref[...] * jnp.dot(h, w_ref[...],
                                          preferred_element_type=jnp.float32)


def _tc_final(a0_ref, a1_ref, g_ref, dinv_ref, b_ref, batch_ref, out_ref):
    p = a0_ref[...] + a1_ref[...] - g_ref[...]
    h = dinv_ref[...] * p + b_ref[...]
    gid = lax.broadcasted_iota(jnp.int32, (G, N_PAD), 0)
    m = (batch_ref[...] == gid).astype(jnp.float32)
    sums = jnp.dot(m, h, preferred_element_type=jnp.float32)
    cnt = jnp.sum(m, axis=1, keepdims=True)
    out_ref[...] = sums / jnp.maximum(cnt, 1.0)


def kernel(x, edge_index, batch, W1, b1, W2, b2, W3, b3):
    f32 = jnp.float32
    src = edge_index[0].astype(jnp.int32)
    dst = edge_index[1].astype(jnp.int32)
    src3 = jnp.concatenate(
        [src, jnp.zeros((E_PAD - E,), jnp.int32)]).reshape(NW, NCHUNK, CH)
    dst3 = jnp.concatenate(
        [dst, jnp.full((E_PAD - E,), N, jnp.int32)]).reshape(NW, NCHUNK, CH)

    x_p = jnp.zeros((N_PAD, IN_DIM), f32).at[:N].set(x)
    batch_p = jnp.full((1, N_PAD), G, jnp.int32).at[0, :N].set(
        batch.astype(jnp.int32))
    z_n = jnp.zeros((N_PAD,), f32)

    dparts = _deg(dst3, z_n)
    dT = dparts.T

    g1, dinv = pl.pallas_call(
        _tc_first,
        out_shape=[jax.ShapeDtypeStruct((N_PAD, HID), f32),
                   jax.ShapeDtypeStruct((N_PAD, 1), f32)],
    )(x_p, W1, dT)

    p1 = _prop64(g1, src3, dst3)
    g2 = pl.pallas_call(
        _tc_mid,
        out_shape=jax.ShapeDtypeStruct((N_PAD, HID), f32),
    )(p1[0], p1[1], g1, dinv, b1.reshape(1, HID), W2)

    p2 = _prop64(g2, src3, dst3)
    g3 = pl.pallas_call(
        _tc_mid,
        out_shape=jax.ShapeDtypeStruct((N_PAD, EMB), f32),
    )(p2[0], p2[1], g2, dinv, b2.reshape(1, HID), W3)

    p3 = _prop32(g3, src3, dst3)
    out = pl.pallas_call(
        _tc_final,
        out_shape=jax.ShapeDtypeStruct((G, EMB), f32),
    )(p3[0], p3[1], g3, dinv, b3.reshape(1, EMB), batch_p)
    return out


# speedup vs baseline: 4.1209x; 1.0335x over previous
import functools

import jax
import jax.numpy as jnp
from jax import lax
from jax.experimental import pallas as pl
from jax.experimental.pallas import tpu as pltpu
from jax.experimental.pallas import tpu_sc as plsc

N = 10000
E = 320000
G = 16
IN_DIM = 128
HID = 64
EMB = 32

NC, NS = 2, 16
NW = NC * NS
CH = 128
NCHUNK = 80
EPT_PAD = NCHUNK * CH
E_PAD = EPT_PAD * NW
N_PAD = 10112
RPT = N_PAD // NS


def _make_prop(D):
    mesh = plsc.VectorSubcoreMesh(core_axis_name="c", subcore_axis_name="s")

    @functools.partial(
        pl.kernel,
        out_type=jax.ShapeDtypeStruct((NC, N_PAD, D), jnp.float32),
        mesh=mesh,
        compiler_params=pltpu.CompilerParams(use_tc_tiling_on_sc=False),
        scratch_types=[
            pltpu.VMEM_SHARED((N_PAD, D), jnp.float32),
            pltpu.VMEM((NCHUNK, CH), jnp.int32),
            pltpu.VMEM((NCHUNK, CH), jnp.int32),
            pltpu.VMEM((CH, D), jnp.float32),
            pltpu.VMEM((CH, D), jnp.float32),
            pltpu.VMEM((CH, D), jnp.float32),
            pltpu.VMEM((CH, D), jnp.float32),
            [pltpu.SemaphoreType.DMA] * 8,
        ],
    )
    def prop(g_hbm, src_hbm, dst_hbm, out_hbm, acc, isrc, idst,
             rows0, rows1, rows2, rows3, sems):
        c = lax.axis_index("c")
        s = lax.axis_index("s")
        wid = c * NS + s
        base = s * RPT
        rows = [rows0, rows1, rows2, rows3]
        gs = [sems[i] for i in range(4)]
        ss = [sems[i + 4] for i in range(4)]
        pltpu.sync_copy(src_hbm.at[wid], isrc)
        pltpu.sync_copy(dst_hbm.at[wid], idst)
        pltpu.sync_copy(g_hbm.at[pl.ds(base, RPT)], acc.at[pl.ds(base, RPT)])
        plsc.subcore_barrier()

        for u in range(4):
            pltpu.async_copy(g_hbm.at[isrc.at[u]], rows[u], gs[u])

        def body(jj, carry):
            j0 = jj * 4
            for u in range(4):
                j = j0 + u
                pltpu.make_async_copy(
                    g_hbm.at[isrc.at[j]], rows[u], gs[u]).wait()
                pltpu.async_copy(rows[u], acc.at[idst.at[j]], ss[u],
                                 add=True)

            @pl.when(jj < NCHUNK // 4 - 1)
            def _():
                for u in range(4):
                    j = j0 + u
                    pltpu.make_async_copy(
                        rows[u], acc.at[idst.at[j]], ss[u]).wait()
                    pltpu.async_copy(g_hbm.at[isrc.at[j + 4]], rows[u],
                                     gs[u])

            @pl.when(jj == NCHUNK // 4 - 1)
            def _():
                for u in range(4):
                    j = j0 + u
                    pltpu.make_async_copy(
                        rows[u], acc.at[idst.at[j]], ss[u]).wait()
            return carry

        lax.fori_loop(0, NCHUNK // 4, body, 0)
        plsc.subcore_barrier()
        pltpu.sync_copy(acc.at[pl.ds(base, RPT)],
                        out_hbm.at[c].at[pl.ds(base, RPT)])

    return prop


_prop64 = _make_prop(HID)
_prop32 = _make_prop(EMB)

def _make_deg():
    mesh = plsc.VectorSubcoreMesh(core_axis_name="c", subcore_axis_name="s")

    @functools.partial(
        pl.kernel,
        out_type=jax.ShapeDtypeStruct((NW, N_PAD), jnp.float32),
        mesh=mesh,
        compiler_params=pltpu.CompilerParams(
            use_tc_tiling_on_sc=False, needs_layout_passes=False),
        scratch_types=[
            pltpu.VMEM((N_PAD,), jnp.float32),
            pltpu.VMEM((NCHUNK, CH), jnp.int32),
        ],
    )
    def deg(dst_hbm, zeros_hbm, out_hbm, acc, idst):
        c = lax.axis_index("c")
        s = lax.axis_index("s")
        wid = c * NS + s
        pltpu.sync_copy(dst_hbm.at[wid], idst)
        pltpu.sync_copy(zeros_hbm, acc)
        onev = jnp.full((16,), 1.0, jnp.float32)

        def body(ci, carry):
            for u in range(CH // 16):
                dstv = idst[ci, pl.ds(u * 16, 16)]
                plsc.addupdate_scatter(acc, [dstv], onev)
            return carry

        lax.fori_loop(0, NCHUNK, body, 0)
        pltpu.sync_copy(acc, out_hbm.at[wid])

    return deg


_deg = _make_deg()


def _tc_first(x_ref, w_ref, dt_ref, g_ref, dinv_ref):
    deg = jnp.sum(dt_ref[...], axis=1, keepdims=True) + 1.0
    dinv = lax.rsqrt(deg)
    dinv_ref[...] = dinv
    g_ref[...] = dinv * jnp.dot(x_ref[...], w_ref[...],
                                preferred_element_type=jnp.float32)


def _tc_mid(a0_ref, a1_ref, g_ref, dinv_ref, b_ref, w_ref, gn_ref):
    p = a0_ref[...] + a1_ref[...] - g_ref[...]
    h = jnp.maximum(dinv_ref[...] * p + b_ref[...], 0.0)
    gn_ref[...] = dinv_ref[...] * jnp.dot(h, w_ref[...],
                                          preferred_element_type=jnp.float32)


def _tc_final(a0_ref, a1_ref, g_ref, dinv_ref, b_ref, batch_ref, out_ref):
    p = a0_ref[...] + a1_ref[...] - g_ref[...]
    h = dinv_ref[...] * p + b_ref[...]
    gid = lax.broadcasted_iota(jnp.int32, (G, N_PAD), 0)
    m = (batch_ref[...] == gid).astype(jnp.float32)
    sums = jnp.dot(m, h, preferred_element_type=jnp.float32)
    cnt = jnp.sum(m, axis=1, keepdims=True)
    out_ref[...] = sums / jnp.maximum(cnt, 1.0)


def kernel(x, edge_index, batch, W1, b1, W2, b2, W3, b3):
    f32 = jnp.float32
    src = edge_index[0].astype(jnp.int32)
    dst = edge_index[1].astype(jnp.int32)
    src3 = jnp.concatenate(
        [src, jnp.zeros((E_PAD - E,), jnp.int32)]).reshape(NW, NCHUNK, CH)
    dst3 = jnp.concatenate(
        [dst, jnp.full((E_PAD - E,), N, jnp.int32)]).reshape(NW, NCHUNK, CH)

    x_p = jnp.zeros((N_PAD, IN_DIM), f32).at[:N].set(x)
    batch_p = jnp.full((1, N_PAD), G, jnp.int32).at[0, :N].set(
        batch.astype(jnp.int32))
    z_n = jnp.zeros((N_PAD,), f32)

    dparts = _deg(dst3, z_n)
    dT = dparts.T

    g1, dinv = pl.pallas_call(
        _tc_first,
        out_shape=[jax.ShapeDtypeStruct((N_PAD, HID), f32),
                   jax.ShapeDtypeStruct((N_PAD, 1), f32)],
    )(x_p, W1, dT)

    p1 = _prop64(g1, src3, dst3)
    g2 = pl.pallas_call(
        _tc_mid,
        out_shape=jax.ShapeDtypeStruct((N_PAD, HID), f32),
    )(p1[0], p1[1], g1, dinv, b1.reshape(1, HID), W2)

    p2 = _prop64(g2, src3, dst3)
    g3 = pl.pallas_call(
        _tc_mid,
        out_shape=jax.ShapeDtypeStruct((N_PAD, EMB), f32),
    )(p2[0], p2[1], g2, dinv, b2.reshape(1, HID), W3)

    p3 = _prop32(g3, src3, dst3)
    out = pl.pallas_call(
        _tc_final,
        out_shape=jax.ShapeDtypeStruct((G, EMB), f32),
    )(p3[0], p3[1], g3, dinv, b3.reshape(1, EMB), batch_p)
    return out
